# baseline (speedup 1.0000x reference)
import functools

import jax
import jax.numpy as jnp
from jax import lax
from jax.experimental import pallas as pl
from jax.experimental.pallas import tpu as pltpu
from jax.experimental.pallas import tpu_sc as plsc

NC = 2
NS = 16
NW = NC * NS
L = 16
B = 16384
D = 128
RPW = B // NW
CH = 64
NCHUNK = RPW // CH
DBLK = 16


_GATHER_DNUMS = lax.GatherDimensionNumbers(
    offset_dims=(), collapsed_slice_dims=(0,), start_index_map=(0,))


def _lane_shuffle(x, perm):
    return lax.gather(x, perm[:, None], dimension_numbers=_GATHER_DNUMS,
                      slice_sizes=(1,),
                      mode=lax.GatherScatterMode.PROMISE_IN_BOUNDS)


def _lanesum(x):
    iota = lax.broadcasted_iota(jnp.int32, (L,), 0)
    for sh in (8, 4, 2, 1):
        x = x + _lane_shuffle(x, iota ^ sh)
    return x


def _tree_sum(xs):
    xs = list(xs)
    while len(xs) > 1:
        xs = [a + b for a, b in zip(xs[0::2], xs[1::2])]
    return xs[0]


def _neg_sqrt(x):
    xs = jnp.maximum(x, jnp.float32(1e-30))
    bits = lax.bitcast_convert_type(xs, jnp.int32)
    y = lax.bitcast_convert_type(jnp.int32(0x5F3759DF) - (bits >> 1),
                                 jnp.float32)
    half = jnp.float32(0.5)
    three_half = jnp.float32(1.5)
    for _ in range(3):
        y = y * (three_half - half * xs * y * y)
    return -(xs * y)


@functools.partial(
    pl.kernel,
    out_type=jax.ShapeDtypeStruct((B,), jnp.float32),
    mesh=plsc.VectorSubcoreMesh(core_axis_name="c", subcore_axis_name="s"),
    compiler_params=pltpu.CompilerParams(needs_layout_passes=False),
    scratch_types=[
        pltpu.VMEM((RPW // 2, 3), jnp.int32),
        pltpu.VMEM((RPW,), jnp.int32),
        pltpu.VMEM((RPW,), jnp.int32),
        pltpu.VMEM((RPW,), jnp.int32),
        pltpu.VMEM((2, CH, D), jnp.float32),
        pltpu.VMEM((2, CH, D), jnp.float32),
        pltpu.VMEM((2, CH, D), jnp.float32),
        pltpu.VMEM((2, CH, D), jnp.float32),
        pltpu.VMEM((RPW,), jnp.float32),
        pltpu.VMEM((L, L), jnp.float32),
        pltpu.SemaphoreType.DMA,
        pltpu.SemaphoreType.DMA,
    ],
)
def _transh_sc(bat, ent, rel, nrm, out,
               bidx, hidx, ridx, tidx, hbuf, tbuf, rbuf, nbuf, outv, svecs,
               sem0, sem1):
    wid = lax.axis_index("s") * NC + lax.axis_index("c")
    base = wid * RPW
    giota = lax.broadcasted_iota(jnp.int32, (L,), 0)
    col0 = jnp.zeros((L,), jnp.int32)
    HALF = RPW // 2
    for half in range(2):
        pltpu.sync_copy(bat.at[pl.ds(base + half * HALF, HALF)], bidx)

        @plsc.parallel_loop(0, HALF // L, 1, unroll=4)
        def deint_body(gg):
            rows16 = gg * L + giota
            dst = half * HALF + gg * L
            hidx[pl.ds(dst, L)] = plsc.load_gather(bidx, [rows16, col0])
            ridx[pl.ds(dst, L)] = plsc.load_gather(bidx, [rows16, col0 + 1])
            tidx[pl.ds(dst, L)] = plsc.load_gather(bidx, [rows16, col0 + 2])

    sems = (sem0, sem1)

    def start_gathers(k, slot):
        off = k * CH
        sem = sems[slot]
        pltpu.async_copy(ent.at[hidx.at[pl.ds(off, CH)]], hbuf.at[slot], sem)
        pltpu.async_copy(ent.at[tidx.at[pl.ds(off, CH)]], tbuf.at[slot], sem)
        pltpu.async_copy(rel.at[ridx.at[pl.ds(off, CH)]], rbuf.at[slot], sem)
        pltpu.async_copy(nrm.at[ridx.at[pl.ds(off, CH)]], nbuf.at[slot], sem)

    def wait_gathers(slot):
        sem = sems[slot]
        dummy = hidx.at[pl.ds(0, CH)]
        pltpu.make_async_copy(ent.at[dummy], hbuf.at[slot], sem).wait()
        pltpu.make_async_copy(ent.at[dummy], tbuf.at[slot], sem).wait()
        pltpu.make_async_copy(rel.at[dummy], rbuf.at[slot], sem).wait()
        pltpu.make_async_copy(nrm.at[dummy], nbuf.at[slot], sem).wait()

    def compute_chunk(k, slot):
        off = k * CH
        hb = hbuf.at[slot]
        tb = tbuf.at[slot]
        rb = rbuf.at[slot]
        nb = nbuf.at[slot]

        iota = lax.broadcasted_iota(jnp.int32, (L,), 0)
        for g in range(CH // L):

            @plsc.parallel_loop(0, L, 1, unroll=4)
            def row_body(rl):
                row = g * L + rl
                prods = []
                for j in range(D // L):
                    h = hb[row, pl.ds(j * L, L)]
                    t = tb[row, pl.ds(j * L, L)]
                    n = nb[row, pl.ds(j * L, L)]
                    prods.append((h - t) * n)
                c = _lanesum(_tree_sum(prods))
                prods2 = []
                for j in range(D // L):
                    h = hb[row, pl.ds(j * L, L)]
                    t = tb[row, pl.ds(j * L, L)]
                    n = nb[row, pl.ds(j * L, L)]
                    r = rb[row, pl.ds(j * L, L)]
                    dvec = (h - t) + r - c * n
                    prods2.append(dvec * dvec)
                svecs[rl, pl.ds(0, L)] = _lanesum(_tree_sum(prods2))

            diag = plsc.load_gather(svecs, [iota, iota])
            outv[pl.ds(off + g * L, L)] = _neg_sqrt(diag)

    start_gathers(0, 0)

    def pair_body(p, carry):
        k0 = p * 2
        wait_gathers(0)
        start_gathers(k0 + 1, 1)
        compute_chunk(k0, 0)
        wait_gathers(1)

        @pl.when(k0 + 2 < NCHUNK)
        def _():
            start_gathers(k0 + 2, 0)

        compute_chunk(k0 + 1, 1)
        return carry

    lax.fori_loop(0, NCHUNK // 2, pair_body, 0)
    pltpu.sync_copy(outv, out.at[pl.ds(base, RPW)])


def kernel(batch, ent_embs, rel_embs, norm_vector):
    score = _transh_sc(batch, ent_embs, rel_embs, norm_vector)
    return score.reshape(-1, 1)


# speedup vs baseline: 1.1701x; 1.1701x over previous
import functools

import jax
import jax.numpy as jnp
from jax import lax
from jax.experimental import pallas as pl
from jax.experimental.pallas import tpu as pltpu
from jax.experimental.pallas import tpu_sc as plsc

NC = 2
NS = 16
NW = NC * NS
L = 16
B = 16384
D = 128
RPW = B // NW
CH = 64
NCHUNK = RPW // CH
DBLK = 16


_GATHER_DNUMS = lax.GatherDimensionNumbers(
    offset_dims=(), collapsed_slice_dims=(0,), start_index_map=(0,))


def _lane_shuffle(x, perm):
    return lax.gather(x, perm[:, None], dimension_numbers=_GATHER_DNUMS,
                      slice_sizes=(1,),
                      mode=lax.GatherScatterMode.PROMISE_IN_BOUNDS)


def _lanesum(x):
    iota = lax.broadcasted_iota(jnp.int32, (L,), 0)
    for sh in (8, 4, 2, 1):
        x = x + _lane_shuffle(x, iota ^ sh)
    return x


def _tree_sum(xs):
    xs = list(xs)
    while len(xs) > 1:
        xs = [a + b for a, b in zip(xs[0::2], xs[1::2])]
    return xs[0]


def _neg_sqrt(x):
    xs = jnp.maximum(x, jnp.float32(1e-30))
    bits = lax.bitcast_convert_type(xs, jnp.int32)
    y = lax.bitcast_convert_type(jnp.int32(0x5F3759DF) - (bits >> 1),
                                 jnp.float32)
    half = jnp.float32(0.5)
    three_half = jnp.float32(1.5)
    for _ in range(3):
        y = y * (three_half - half * xs * y * y)
    return -(xs * y)


@functools.partial(
    pl.kernel,
    out_type=jax.ShapeDtypeStruct((B,), jnp.float32),
    mesh=plsc.VectorSubcoreMesh(core_axis_name="c", subcore_axis_name="s"),
    compiler_params=pltpu.CompilerParams(needs_layout_passes=False),
    scratch_types=[
        pltpu.VMEM((RPW // 2, 3), jnp.int32),
        pltpu.VMEM((RPW,), jnp.int32),
        pltpu.VMEM((RPW,), jnp.int32),
        pltpu.VMEM((RPW,), jnp.int32),
        pltpu.VMEM((2, CH, D), jnp.float32),
        pltpu.VMEM((2, CH, D), jnp.float32),
        pltpu.VMEM((2, CH, D), jnp.float32),
        pltpu.VMEM((2, CH, D), jnp.float32),
        pltpu.VMEM((RPW,), jnp.float32),
        pltpu.SemaphoreType.DMA,
        pltpu.SemaphoreType.DMA,
    ],
)
def _transh_sc(bat, ent, rel, nrm, out,
               bidx, hidx, ridx, tidx, hbuf, tbuf, rbuf, nbuf, outv,
               sem0, sem1):
    wid = lax.axis_index("s") * NC + lax.axis_index("c")
    base = wid * RPW
    giota = lax.broadcasted_iota(jnp.int32, (L,), 0)
    col0 = jnp.zeros((L,), jnp.int32)
    HALF = RPW // 2
    for half in range(2):
        pltpu.sync_copy(bat.at[pl.ds(base + half * HALF, HALF)], bidx)

        @plsc.parallel_loop(0, HALF // L, 1, unroll=4)
        def deint_body(gg):
            rows16 = gg * L + giota
            dst = half * HALF + gg * L
            hidx[pl.ds(dst, L)] = plsc.load_gather(bidx, [rows16, col0])
            ridx[pl.ds(dst, L)] = plsc.load_gather(bidx, [rows16, col0 + 1])
            tidx[pl.ds(dst, L)] = plsc.load_gather(bidx, [rows16, col0 + 2])

    sems = (sem0, sem1)

    def start_gathers(k, slot):
        off = k * CH
        sem = sems[slot]
        pltpu.async_copy(ent.at[hidx.at[pl.ds(off, CH)]], hbuf.at[slot], sem)
        pltpu.async_copy(ent.at[tidx.at[pl.ds(off, CH)]], tbuf.at[slot], sem)
        pltpu.async_copy(rel.at[ridx.at[pl.ds(off, CH)]], rbuf.at[slot], sem)
        pltpu.async_copy(nrm.at[ridx.at[pl.ds(off, CH)]], nbuf.at[slot], sem)

    def wait_gathers(slot):
        sem = sems[slot]
        dummy = hidx.at[pl.ds(0, CH)]
        pltpu.make_async_copy(ent.at[dummy], hbuf.at[slot], sem).wait()
        pltpu.make_async_copy(ent.at[dummy], tbuf.at[slot], sem).wait()
        pltpu.make_async_copy(rel.at[dummy], rbuf.at[slot], sem).wait()
        pltpu.make_async_copy(nrm.at[dummy], nbuf.at[slot], sem).wait()

    def compute_chunk(k, slot):
        off = k * CH
        hb = hbuf.at[slot]
        tb = tbuf.at[slot]
        rb = rbuf.at[slot]
        nb = nbuf.at[slot]

        iota = lax.broadcasted_iota(jnp.int32, (L,), 0)
        lane0 = iota == 0

        @plsc.parallel_loop(0, CH, 1, unroll=4)
        def row_body(row):
            prods = []
            for j in range(D // L):
                h = hb[row, pl.ds(j * L, L)]
                t = tb[row, pl.ds(j * L, L)]
                n = nb[row, pl.ds(j * L, L)]
                prods.append((h - t) * n)
            c = _lanesum(_tree_sum(prods))
            prods2 = []
            for j in range(D // L):
                h = hb[row, pl.ds(j * L, L)]
                t = tb[row, pl.ds(j * L, L)]
                n = nb[row, pl.ds(j * L, L)]
                r = rb[row, pl.ds(j * L, L)]
                dvec = (h - t) + r - c * n
                prods2.append(dvec * dvec)
            s = _lanesum(_tree_sum(prods2))
            plsc.store_scatter(outv, [iota * 0 + (off + row)], s,
                               mask=lane0)

    start_gathers(0, 0)

    def pair_body(p, carry):
        k0 = p * 2
        wait_gathers(0)
        start_gathers(k0 + 1, 1)
        compute_chunk(k0, 0)
        wait_gathers(1)

        @pl.when(k0 + 2 < NCHUNK)
        def _():
            start_gathers(k0 + 2, 0)

        compute_chunk(k0 + 1, 1)
        return carry

    lax.fori_loop(0, NCHUNK // 2, pair_body, 0)

    @plsc.parallel_loop(0, RPW // L, 1, unroll=4)
    def sqrt_body(gg):
        outv[pl.ds(gg * L, L)] = _neg_sqrt(outv[pl.ds(gg * L, L)])

    pltpu.sync_copy(outv, out.at[pl.ds(base, RPW)])


def kernel(batch, ent_embs, rel_embs, norm_vector):
    score = _transh_sc(batch, ent_embs, rel_embs, norm_vector)
    return score.reshape(-1, 1)


# speedup vs baseline: 1.3314x; 1.1379x over previous
import functools

import jax
import jax.numpy as jnp
from jax import lax
from jax.experimental import pallas as pl
from jax.experimental.pallas import tpu as pltpu
from jax.experimental.pallas import tpu_sc as plsc

NC = 2
NS = 16
NW = NC * NS
L = 16
B = 16384
D = 128
RPW = B // NW
CH = 64
NCHUNK = RPW // CH

_GATHER_DNUMS = lax.GatherDimensionNumbers(
    offset_dims=(), collapsed_slice_dims=(0,), start_index_map=(0,))


def _lane_shuffle(x, perm):
    return lax.gather(x, perm[:, None], dimension_numbers=_GATHER_DNUMS,
                      slice_sizes=(1,),
                      mode=lax.GatherScatterMode.PROMISE_IN_BOUNDS)


def _lanesum(x):
    iota = lax.broadcasted_iota(jnp.int32, (L,), 0)
    for sh in (8, 4, 2, 1):
        x = x + _lane_shuffle(x, iota ^ sh)
    return x


def _tree_sum(xs):
    xs = list(xs)
    while len(xs) > 1:
        xs = [a + b for a, b in zip(xs[0::2], xs[1::2])]
    return xs[0]


def _neg_sqrt(x):
    xs = jnp.maximum(x, jnp.float32(1e-30))
    bits = lax.bitcast_convert_type(xs, jnp.int32)
    y = lax.bitcast_convert_type(jnp.int32(0x5F3759DF) - (bits >> 1),
                                 jnp.float32)
    half = jnp.float32(0.5)
    three_half = jnp.float32(1.5)
    for _ in range(3):
        y = y * (three_half - half * xs * y * y)
    return -(xs * y)


@functools.partial(
    pl.kernel,
    out_type=jax.ShapeDtypeStruct((B,), jnp.float32),
    mesh=plsc.VectorSubcoreMesh(core_axis_name="c", subcore_axis_name="s"),
    compiler_params=pltpu.CompilerParams(needs_layout_passes=False),
    scratch_types=[
        pltpu.VMEM((RPW,), jnp.int32),
        pltpu.VMEM((RPW,), jnp.int32),
        pltpu.VMEM((RPW,), jnp.int32),
        pltpu.VMEM((2, CH, D), jnp.float32),
        pltpu.VMEM((2, CH, D), jnp.float32),
        pltpu.VMEM((2, CH, D), jnp.float32),
        pltpu.VMEM((2, CH, D), jnp.float32),
        pltpu.VMEM((RPW,), jnp.float32),
        pltpu.VMEM((CH, L), jnp.float32),
        pltpu.SemaphoreType.DMA,
        pltpu.SemaphoreType.DMA,
    ],
)
def _transh_sc(batT, ent, rel, nrm, out,
               hidx, ridx, tidx, hbuf, tbuf, rbuf, nbuf, outv, svecs,
               sem0, sem1):
    wid = lax.axis_index("s") * NC + lax.axis_index("c")
    base = wid * RPW
    pltpu.sync_copy(batT.at[pl.ds(base, RPW)], hidx)
    pltpu.sync_copy(batT.at[pl.ds(B + base, RPW)], ridx)
    pltpu.sync_copy(batT.at[pl.ds(2 * B + base, RPW)], tidx)

    sems = (sem0, sem1)

    def start_gathers(k, slot):
        off = k * CH
        sem = sems[slot]
        pltpu.async_copy(ent.at[hidx.at[pl.ds(off, CH)]], hbuf.at[slot], sem)
        pltpu.async_copy(ent.at[tidx.at[pl.ds(off, CH)]], tbuf.at[slot], sem)
        pltpu.async_copy(rel.at[ridx.at[pl.ds(off, CH)]], rbuf.at[slot], sem)
        pltpu.async_copy(nrm.at[ridx.at[pl.ds(off, CH)]], nbuf.at[slot], sem)

    def wait_gathers(slot):
        sem = sems[slot]
        dummy = hidx.at[pl.ds(0, CH)]
        pltpu.make_async_copy(ent.at[dummy], hbuf.at[slot], sem).wait()
        pltpu.make_async_copy(ent.at[dummy], tbuf.at[slot], sem).wait()
        pltpu.make_async_copy(rel.at[dummy], rbuf.at[slot], sem).wait()
        pltpu.make_async_copy(nrm.at[dummy], nbuf.at[slot], sem).wait()

    def compute_chunk(k, slot):
        off = k * CH
        hb = hbuf.at[slot]
        tb = tbuf.at[slot]
        rb = rbuf.at[slot]
        nb = nbuf.at[slot]

        @plsc.parallel_loop(0, CH, 1, unroll=4)
        def row_body(row):
            prods = []
            for j in range(D // L):
                h = hb[row, pl.ds(j * L, L)]
                t = tb[row, pl.ds(j * L, L)]
                n = nb[row, pl.ds(j * L, L)]
                prods.append((h - t) * n)
            c = _lanesum(_tree_sum(prods))
            prods2 = []
            for j in range(D // L):
                h = hb[row, pl.ds(j * L, L)]
                t = tb[row, pl.ds(j * L, L)]
                n = nb[row, pl.ds(j * L, L)]
                r = rb[row, pl.ds(j * L, L)]
                dvec = (h - t) + r - c * n
                prods2.append(dvec * dvec)
            svecs[row, pl.ds(0, L)] = _lanesum(_tree_sum(prods2))

        iota = lax.broadcasted_iota(jnp.int32, (L,), 0)
        for g in range(CH // L):
            diag = plsc.load_gather(svecs, [g * L + iota, iota])
            outv[pl.ds(off + g * L, L)] = _neg_sqrt(diag)

    start_gathers(0, 0)

    def pair_body(p, carry):
        k0 = p * 2
        wait_gathers(0)
        start_gathers(k0 + 1, 1)
        compute_chunk(k0, 0)
        wait_gathers(1)

        @pl.when(k0 + 2 < NCHUNK)
        def _():
            start_gathers(k0 + 2, 0)

        compute_chunk(k0 + 1, 1)
        return carry

    lax.fori_loop(0, NCHUNK // 2, pair_body, 0)
    pltpu.sync_copy(outv, out.at[pl.ds(base, RPW)])


def kernel(batch, ent_embs, rel_embs, norm_vector):
    score = _transh_sc(batch.T.reshape(-1), ent_embs, rel_embs, norm_vector)
    return score.reshape(-1, 1)
